# baseline (speedup 1.0000x reference)
import functools
import jax
import jax.numpy as jnp
from jax import lax
from jax.experimental import pallas as pl
from jax.experimental.pallas import tpu as pltpu
from jax.experimental.pallas import tpu_sc as plsc

N_NODE = 1000000
N_DIM = 64
BATCH = 16384

NC = 2
NS = 16
NW = NC * NS
B_PER_W = BATCH // NW
WBLK = 1
NBUF = 7
WNODES = WBLK * 128
NBLOCKS = (N_NODE + 127) // 128
MAX_WSTART = NBLOCKS - WBLK
GCHUNK = 128


def _extract_one_table(tbl_hbm, nodes, winA, stage, sem):
    lanes = lax.iota(jnp.int32, 16)

    def win_src(ws):
        wc = jnp.minimum(ws, MAX_WSTART)
        off = pl.multiple_of(wc * 128, 128)
        return wc, tbl_hbm.at[pl.ds(0, N_DIM), pl.ds(off, WNODES)]

    def node_at(cur):
        return nodes[pl.ds(cur, 16)][0]

    def extract(buf, ws, cursor):
        wc = jnp.minimum(ws, MAX_WSTART)
        wend = (wc + WBLK) * 128

        def cond(cur):
            return jnp.logical_and(cur < B_PER_W, node_at(cur) < wend)

        def body(cur):
            nl = node_at(cur) - wc * 128
            nlv = jnp.full((16,), nl, jnp.int32)
            for c in range(N_DIM // 16):
                g = plsc.load_gather(buf, [lanes + c * 16, nlv])
                stage[cur, pl.ds(c * 16, 16)] = g
            return cur + 1

        return lax.while_loop(cond, body, cursor)

    first = lax.shift_right_logical(node_at(0), 7)
    bufs = [winA.at[b] for b in range(NBUF)]
    for b in range(NBUF):
        pltpu.async_copy(win_src(first + b)[1], bufs[b], sem)

    def outer_cond(state):
        ws, cursor = state
        return cursor < B_PER_W

    def outer_body(state):
        ws, cursor = state
        for b in range(NBUF):
            pltpu.make_async_copy(win_src(ws + b)[1], bufs[b], sem).wait()
            cursor = extract(bufs[b], ws + b, cursor)
            pltpu.async_copy(win_src(ws + NBUF + b)[1], bufs[b], sem)
        return (ws + NBUF, cursor)

    ws_end, _ = lax.while_loop(outer_cond, outer_body, (first, jnp.int32(0)))
    for b in range(NBUF):
        pltpu.make_async_copy(win_src(ws_end + b)[1], bufs[b], sem).wait()


def _extract_kernel(us_hbm, vs_hbm, node_hbm, ctx_hbm, eu_hbm, ev_hbm,
                    nodes, winA, stage, sem):
    wid = lax.axis_index("s") * NC + lax.axis_index("c")
    base = wid * B_PER_W

    pltpu.sync_copy(us_hbm.at[pl.ds(base, B_PER_W)], nodes.at[pl.ds(0, B_PER_W)])
    _extract_one_table(node_hbm, nodes, winA, stage, sem)
    pltpu.sync_copy(stage, eu_hbm.at[pl.ds(base, B_PER_W)])

    pltpu.sync_copy(vs_hbm.at[pl.ds(base, B_PER_W)], nodes.at[pl.ds(0, B_PER_W)])
    _extract_one_table(ctx_hbm, nodes, winA, stage, sem)
    pltpu.sync_copy(stage, ev_hbm.at[pl.ds(base, B_PER_W)])


def _dot_kernel(iu_hbm, iv_hbm, eu_hbm, ev_hbm, out_hbm,
                idx_u, idx_v, rows_u, rows_v, out_v, sem):
    wid = lax.axis_index("s") * NC + lax.axis_index("c")
    base = wid * B_PER_W

    pltpu.sync_copy(iu_hbm.at[pl.ds(base, B_PER_W)], idx_u)
    pltpu.sync_copy(iv_hbm.at[pl.ds(base, B_PER_W)], idx_v)

    lanes = lax.iota(jnp.int32, 16)
    nch = B_PER_W // GCHUNK

    def fire(ci, bu, bv):
        coff = ci * GCHUNK
        pltpu.async_copy(eu_hbm.at[idx_u.at[pl.ds(coff, GCHUNK)]], bu, sem)
        pltpu.async_copy(ev_hbm.at[idx_v.at[pl.ds(coff, GCHUNK)]], bv, sem)

    def drain(ci, bu, bv):
        coff = ci * GCHUNK
        pltpu.make_async_copy(eu_hbm.at[idx_u.at[pl.ds(coff, GCHUNK)]], bu, sem).wait()
        pltpu.make_async_copy(ev_hbm.at[idx_v.at[pl.ds(coff, GCHUNK)]], bv, sem).wait()

    fire(0, rows_u.at[0], rows_v.at[0])
    fire(1, rows_u.at[1], rows_v.at[1])
    for ci in range(nch):
        bu, bv = rows_u.at[ci % 2], rows_v.at[ci % 2]
        drain(ci, bu, bv)
        coff = ci * GCHUNK
        for sb in range(GCHUNK // 16):
            acc = jnp.zeros((16,), jnp.float32)
            for i in range(16):
                e = sb * 16 + i
                t = bu[e, pl.ds(0, 16)] * bv[e, pl.ds(0, 16)]
                for c in range(1, N_DIM // 16):
                    t += bu[e, pl.ds(c * 16, 16)] * bv[e, pl.ds(c * 16, 16)]
                acc = jnp.where(lanes == i, jnp.sum(t), acc)
            out_v[pl.ds(coff + sb * 16, 16)] = acc
        if ci + 2 < nch:
            fire(ci + 2, bu, bv)

    pltpu.sync_copy(out_v, out_hbm.at[pl.ds(base, B_PER_W)])


@jax.jit
def kernel(u, v, node_embed, context_node_embed):
    iota = lax.iota(jnp.int32, BATCH)
    u_s, pu = lax.sort_key_val(u, iota)
    v_s, pv = lax.sort_key_val(v, iota)
    inv_pu = jnp.zeros((BATCH,), jnp.int32).at[pu].set(iota)
    inv_pv = jnp.zeros((BATCH,), jnp.int32).at[pv].set(iota)

    mesh = plsc.VectorSubcoreMesh(core_axis_name="c", subcore_axis_name="s")
    params = pltpu.CompilerParams(needs_layout_passes=False)

    extract = functools.partial(
        pl.kernel,
        out_type=(jax.ShapeDtypeStruct((BATCH, 128), jnp.float32),
                  jax.ShapeDtypeStruct((BATCH, 128), jnp.float32)),
        mesh=mesh,
        compiler_params=params,
        scratch_types=[
            pltpu.VMEM((B_PER_W + 16,), jnp.int32),
            pltpu.VMEM((NBUF, N_DIM, WNODES), jnp.float32),
            pltpu.VMEM((B_PER_W, 128), jnp.float32),
            pltpu.SemaphoreType.DMA,
        ],
    )(_extract_kernel)
    eu, ev = extract(u_s, v_s, node_embed.T, context_node_embed.T)

    dot = functools.partial(
        pl.kernel,
        out_type=jax.ShapeDtypeStruct((BATCH,), jnp.float32),
        mesh=mesh,
        compiler_params=params,
        scratch_types=[
            pltpu.VMEM((B_PER_W,), jnp.int32),
            pltpu.VMEM((B_PER_W,), jnp.int32),
            pltpu.VMEM((2, GCHUNK, 128), jnp.float32),
            pltpu.VMEM((2, GCHUNK, 128), jnp.float32),
            pltpu.VMEM((B_PER_W,), jnp.float32),
            pltpu.SemaphoreType.DMA,
        ],
    )(_dot_kernel)
    return dot(inv_pu, inv_pv, eu, ev)


# speedup vs baseline: 1.0465x; 1.0465x over previous
import functools
import jax
import jax.numpy as jnp
from jax import lax
from jax.experimental import pallas as pl
from jax.experimental.pallas import tpu as pltpu
from jax.experimental.pallas import tpu_sc as plsc

N_NODE = 1000000
N_DIM = 64
BATCH = 16384

NC = 2
NS = 16
NW = NC * NS
B_PER_W = BATCH // NW
WBLK = 1
NBUF = 7
WNODES = WBLK * 128
NBLOCKS = (N_NODE + 127) // 128
MAX_WSTART = NBLOCKS - WBLK
GCHUNK = 128


def _extract_one_table(tbl_hbm, nodes, winA, stage, sem):
    lanes = lax.iota(jnp.int32, 16)

    def win_src(ws):
        wc = jnp.minimum(ws, MAX_WSTART)
        off = pl.multiple_of(wc * 128, 128)
        return wc, tbl_hbm.at[pl.ds(0, N_DIM), pl.ds(off, WNODES)]

    def node_at(cur):
        return nodes[pl.ds(cur, 16)][0]

    def extract(buf, ws, cursor):
        wc = jnp.minimum(ws, MAX_WSTART)
        wend = (wc + WBLK) * 128

        def cond(cur):
            return jnp.logical_and(cur < B_PER_W, node_at(cur) < wend)

        def body(cur):
            nl = node_at(cur) - wc * 128
            nlv = jnp.full((16,), nl, jnp.int32)
            for c in range(N_DIM // 16):
                g = plsc.load_gather(buf, [lanes + c * 16, nlv])
                stage[cur, pl.ds(c * 16, 16)] = g
            return cur + 1

        return lax.while_loop(cond, body, cursor)

    first = lax.shift_right_logical(node_at(0), 7)
    bufs = [winA.at[b] for b in range(NBUF)]
    for b in range(NBUF):
        pltpu.async_copy(win_src(first + b)[1], bufs[b], sem)

    def outer_cond(state):
        ws, cursor = state
        return cursor < B_PER_W

    def outer_body(state):
        ws, cursor = state
        for b in range(NBUF):
            pltpu.make_async_copy(win_src(ws + b)[1], bufs[b], sem).wait()
            cursor = extract(bufs[b], ws + b, cursor)
            pltpu.async_copy(win_src(ws + NBUF + b)[1], bufs[b], sem)
        return (ws + NBUF, cursor)

    ws_end, _ = lax.while_loop(outer_cond, outer_body, (first, jnp.int32(0)))
    for b in range(NBUF):
        pltpu.make_async_copy(win_src(ws_end + b)[1], bufs[b], sem).wait()


def _extract_kernel(us_hbm, vs_hbm, node_hbm, ctx_hbm, eu_hbm, ev_hbm,
                    nodes, winA, stage, sem):
    wid = lax.axis_index("s") * NC + lax.axis_index("c")
    base = wid * B_PER_W

    pltpu.sync_copy(us_hbm.at[pl.ds(base, B_PER_W)], nodes.at[pl.ds(0, B_PER_W)])
    _extract_one_table(node_hbm, nodes, winA, stage, sem)
    pltpu.sync_copy(stage, eu_hbm.at[pl.ds(base, B_PER_W)])

    pltpu.sync_copy(vs_hbm.at[pl.ds(base, B_PER_W)], nodes.at[pl.ds(0, B_PER_W)])
    _extract_one_table(ctx_hbm, nodes, winA, stage, sem)
    pltpu.sync_copy(stage, ev_hbm.at[pl.ds(base, B_PER_W)])


def _dot_kernel(iu_hbm, iv_hbm, eu_hbm, ev_hbm, out_hbm,
                idx_u, idx_v, rows_u, rows_v, out_v, sem):
    wid = lax.axis_index("s") * NC + lax.axis_index("c")
    base = wid * B_PER_W

    pltpu.sync_copy(iu_hbm.at[pl.ds(base, B_PER_W)], idx_u)
    pltpu.sync_copy(iv_hbm.at[pl.ds(base, B_PER_W)], idx_v)

    lanes = lax.iota(jnp.int32, 16)

    def chunk_body(ci, _):
        coff = ci * GCHUNK
        cu = pltpu.async_copy(eu_hbm.at[idx_u.at[pl.ds(coff, GCHUNK)]], rows_u, sem)
        cv = pltpu.async_copy(ev_hbm.at[idx_v.at[pl.ds(coff, GCHUNK)]], rows_v, sem)
        cu.wait()
        cv.wait()
        for sb in range(GCHUNK // 16):
            acc = jnp.zeros((16,), jnp.float32)
            for i in range(16):
                e = sb * 16 + i
                t = rows_u[e, pl.ds(0, 16)] * rows_v[e, pl.ds(0, 16)]
                for c in range(1, N_DIM // 16):
                    t += rows_u[e, pl.ds(c * 16, 16)] * rows_v[e, pl.ds(c * 16, 16)]
                acc = jnp.where(lanes == i, jnp.sum(t), acc)
            out_v[pl.ds(coff + sb * 16, 16)] = acc
        return ()

    lax.fori_loop(0, B_PER_W // GCHUNK, chunk_body, ())

    pltpu.sync_copy(out_v, out_hbm.at[pl.ds(base, B_PER_W)])


@jax.jit
def kernel(u, v, node_embed, context_node_embed):
    iota = lax.iota(jnp.int32, BATCH)
    u_s, pu = lax.sort_key_val(u, iota)
    v_s, pv = lax.sort_key_val(v, iota)
    inv_pu = jnp.zeros((BATCH,), jnp.int32).at[pu].set(iota)
    inv_pv = jnp.zeros((BATCH,), jnp.int32).at[pv].set(iota)

    mesh = plsc.VectorSubcoreMesh(core_axis_name="c", subcore_axis_name="s")
    params = pltpu.CompilerParams(needs_layout_passes=False)

    extract = functools.partial(
        pl.kernel,
        out_type=(jax.ShapeDtypeStruct((BATCH, 128), jnp.float32),
                  jax.ShapeDtypeStruct((BATCH, 128), jnp.float32)),
        mesh=mesh,
        compiler_params=params,
        scratch_types=[
            pltpu.VMEM((B_PER_W + 16,), jnp.int32),
            pltpu.VMEM((NBUF, N_DIM, WNODES), jnp.float32),
            pltpu.VMEM((B_PER_W, 128), jnp.float32),
            pltpu.SemaphoreType.DMA,
        ],
    )(_extract_kernel)
    eu, ev = extract(u_s, v_s, node_embed.T, context_node_embed.T)

    dot = functools.partial(
        pl.kernel,
        out_type=jax.ShapeDtypeStruct((BATCH,), jnp.float32),
        mesh=mesh,
        compiler_params=params,
        scratch_types=[
            pltpu.VMEM((B_PER_W,), jnp.int32),
            pltpu.VMEM((B_PER_W,), jnp.int32),
            pltpu.VMEM((GCHUNK, 128), jnp.float32),
            pltpu.VMEM((GCHUNK, 128), jnp.float32),
            pltpu.VMEM((B_PER_W,), jnp.float32),
            pltpu.SemaphoreType.DMA,
        ],
    )(_dot_kernel)
    return dot(inv_pu, inv_pv, eu, ev)


# speedup vs baseline: 1.1020x; 1.0531x over previous
import functools
import jax
import jax.numpy as jnp
from jax import lax
from jax.experimental import pallas as pl
from jax.experimental.pallas import tpu as pltpu
from jax.experimental.pallas import tpu_sc as plsc

N_NODE = 1000000
N_DIM = 64
BATCH = 16384

NC = 2
NS = 16
NW = NC * NS
B_PER_W = BATCH // NW
WBLK = 1
NBUF = 7
WNODES = WBLK * 128
NBLOCKS = (N_NODE + 127) // 128
MAX_WSTART = NBLOCKS - WBLK
GCHUNK = 128


def _extract_one_table(tbl_hbm, nodes, winA, stage, blist, sem):
    lanes = lax.iota(jnp.int32, 16)

    def node_at(cur):
        return nodes[pl.ds(cur, 16)][0]

    def bcond(st):
        i, j, prev = st
        return i < B_PER_W

    def bbody(st):
        i, j, prev = st
        b = lax.shift_right_logical(node_at(i), 7)
        new = b != prev

        @pl.when(new)
        def _():
            blist[j] = b

        return (i + 1, jnp.where(new, j + 1, j), b)

    _, cnt, _ = lax.while_loop(
        bcond, bbody, (jnp.int32(0), jnp.int32(0), jnp.int32(-1)))

    def blk_at(k):
        return blist[jnp.minimum(k, cnt - 1)]

    def win_src(k):
        off = pl.multiple_of(blk_at(k) * 128, 128)
        return tbl_hbm.at[pl.ds(0, N_DIM), pl.ds(off, WNODES)]

    def extract(buf, k, cursor):
        wc = blk_at(k)
        wend = (wc + WBLK) * 128

        def cond(cur):
            return jnp.logical_and(cur < B_PER_W, node_at(cur) < wend)

        def body(cur):
            nl = node_at(cur) - wc * 128
            nlv = jnp.full((16,), nl, jnp.int32)
            for c in range(N_DIM // 16):
                g = plsc.load_gather(buf, [lanes + c * 16, nlv])
                stage[cur, pl.ds(c * 16, 16)] = g
            return cur + 1

        return lax.while_loop(cond, body, cursor)

    bufs = [winA.at[b] for b in range(NBUF)]
    for b in range(NBUF):
        pltpu.async_copy(win_src(b), bufs[b], sem)

    def outer_cond(state):
        ws, cursor = state
        return cursor < B_PER_W

    def outer_body(state):
        ws, cursor = state
        for b in range(NBUF):
            pltpu.make_async_copy(win_src(ws + b), bufs[b], sem).wait()
            cursor = extract(bufs[b], ws + b, cursor)
            pltpu.async_copy(win_src(ws + NBUF + b), bufs[b], sem)
        return (ws + NBUF, cursor)

    ws_end, _ = lax.while_loop(outer_cond, outer_body,
                               (jnp.int32(0), jnp.int32(0)))
    for b in range(NBUF):
        pltpu.make_async_copy(win_src(ws_end + b), bufs[b], sem).wait()


def _extract_kernel(us_hbm, vs_hbm, node_hbm, ctx_hbm, eu_hbm, ev_hbm,
                    nodes, winA, stage, blist, sem):
    wid = lax.axis_index("s") * NC + lax.axis_index("c")
    base = wid * B_PER_W

    pltpu.sync_copy(us_hbm.at[pl.ds(base, B_PER_W)], nodes.at[pl.ds(0, B_PER_W)])
    _extract_one_table(node_hbm, nodes, winA, stage, blist, sem)
    pltpu.sync_copy(stage, eu_hbm.at[pl.ds(base, B_PER_W)])

    pltpu.sync_copy(vs_hbm.at[pl.ds(base, B_PER_W)], nodes.at[pl.ds(0, B_PER_W)])
    _extract_one_table(ctx_hbm, nodes, winA, stage, blist, sem)
    pltpu.sync_copy(stage, ev_hbm.at[pl.ds(base, B_PER_W)])


def _dot_kernel(iu_hbm, iv_hbm, eu_hbm, ev_hbm, out_hbm,
                idx_u, idx_v, rows_u, rows_v, out_v, sem):
    wid = lax.axis_index("s") * NC + lax.axis_index("c")
    base = wid * B_PER_W

    pltpu.sync_copy(iu_hbm.at[pl.ds(base, B_PER_W)], idx_u)
    pltpu.sync_copy(iv_hbm.at[pl.ds(base, B_PER_W)], idx_v)

    lanes = lax.iota(jnp.int32, 16)

    def chunk_body(ci, _):
        coff = ci * GCHUNK
        cu = pltpu.async_copy(eu_hbm.at[idx_u.at[pl.ds(coff, GCHUNK)]], rows_u, sem)
        cv = pltpu.async_copy(ev_hbm.at[idx_v.at[pl.ds(coff, GCHUNK)]], rows_v, sem)
        cu.wait()
        cv.wait()
        for sb in range(GCHUNK // 16):
            acc = jnp.zeros((16,), jnp.float32)
            for i in range(16):
                e = sb * 16 + i
                t = rows_u[e, pl.ds(0, 16)] * rows_v[e, pl.ds(0, 16)]
                for c in range(1, N_DIM // 16):
                    t += rows_u[e, pl.ds(c * 16, 16)] * rows_v[e, pl.ds(c * 16, 16)]
                acc = jnp.where(lanes == i, jnp.sum(t), acc)
            out_v[pl.ds(coff + sb * 16, 16)] = acc
        return ()

    lax.fori_loop(0, B_PER_W // GCHUNK, chunk_body, ())

    pltpu.sync_copy(out_v, out_hbm.at[pl.ds(base, B_PER_W)])


@jax.jit
def kernel(u, v, node_embed, context_node_embed):
    iota = lax.iota(jnp.int32, BATCH)
    u_s, pu = lax.sort_key_val(u, iota)
    v_s, pv = lax.sort_key_val(v, iota)
    inv_pu = jnp.zeros((BATCH,), jnp.int32).at[pu].set(iota)
    inv_pv = jnp.zeros((BATCH,), jnp.int32).at[pv].set(iota)

    mesh = plsc.VectorSubcoreMesh(core_axis_name="c", subcore_axis_name="s")
    params = pltpu.CompilerParams(needs_layout_passes=False)

    extract = functools.partial(
        pl.kernel,
        out_type=(jax.ShapeDtypeStruct((BATCH, 128), jnp.float32),
                  jax.ShapeDtypeStruct((BATCH, 128), jnp.float32)),
        mesh=mesh,
        compiler_params=params,
        scratch_types=[
            pltpu.VMEM((B_PER_W + 16,), jnp.int32),
            pltpu.VMEM((NBUF, N_DIM, WNODES), jnp.float32),
            pltpu.VMEM((B_PER_W, 128), jnp.float32),
            pltpu.SMEM((B_PER_W,), jnp.int32),
            pltpu.SemaphoreType.DMA,
        ],
    )(_extract_kernel)
    eu, ev = extract(u_s, v_s, node_embed.T, context_node_embed.T)

    dot = functools.partial(
        pl.kernel,
        out_type=jax.ShapeDtypeStruct((BATCH,), jnp.float32),
        mesh=mesh,
        compiler_params=params,
        scratch_types=[
            pltpu.VMEM((B_PER_W,), jnp.int32),
            pltpu.VMEM((B_PER_W,), jnp.int32),
            pltpu.VMEM((GCHUNK, 128), jnp.float32),
            pltpu.VMEM((GCHUNK, 128), jnp.float32),
            pltpu.VMEM((B_PER_W,), jnp.float32),
            pltpu.SemaphoreType.DMA,
        ],
    )(_dot_kernel)
    return dot(inv_pu, inv_pv, eu, ev)
